# Initial kernel scaffold; baseline (speedup 1.0000x reference)
import jax
import jax.numpy as jnp
from jax.experimental import pallas as pl


def kernel(x, edge_index, W1s, W1d, a1s, a1d, b1, Wl1, bl1, W2s, W2d, a2s, a2d, b2, Wl2, bl2):
    raise NotImplementedError("write your pallas kernel here")


# speedup vs baseline: 4.8792x; 4.8792x over previous
import functools
import jax
import jax.numpy as jnp
from jax.experimental import pallas as pl
from jax.experimental.pallas import tpu as pltpu

_N, _E, _D, _HID, _OUT, _H = 10000, 160000, 256, 256, 256, 4
_NB = 10
_BR = _N // _NB


def _dense1_body(x_ref, Ws_ref, Vs_ref, Vd_ref, Wl_ref,
                 xs_ref, es_ref, ed_ref, skip_ref):
    x = x_ref[...]
    xs_ref[...] = jnp.dot(x, Ws_ref[...], preferred_element_type=jnp.float32)
    es_ref[...] = jnp.dot(x, Vs_ref[...], preferred_element_type=jnp.float32)
    ed_ref[...] = jnp.dot(x, Vd_ref[...], preferred_element_type=jnp.float32)
    skip_ref[...] = jnp.dot(x, Wl_ref[...], preferred_element_type=jnp.float32)


def _dense2_body(msg_ref, skip_ref, b_ref, Ws_ref, Vs_ref, Vd_ref, Wl_ref,
                 h_ref, xs_ref, es_ref, ed_ref, skip2_ref):
    m = msg_ref[...]
    hid = m.shape[-1] // _H
    acc = m[:, :hid] + m[:, hid:2 * hid] + m[:, 2 * hid:3 * hid] + m[:, 3 * hid:]
    h = jnp.maximum(acc * (1.0 / _H) + b_ref[...] + skip_ref[...], 0.0)
    h_ref[...] = h
    xs_ref[...] = jnp.dot(h, Ws_ref[...], preferred_element_type=jnp.float32)
    es_ref[...] = jnp.dot(h, Vs_ref[...], preferred_element_type=jnp.float32)
    ed_ref[...] = jnp.dot(h, Vd_ref[...], preferred_element_type=jnp.float32)
    skip2_ref[...] = jnp.dot(h, Wl_ref[...], preferred_element_type=jnp.float32)


def _final_body(msg_ref, skip_ref, b_ref, o_ref):
    m = msg_ref[...]
    out = m.shape[-1] // _H
    acc = m[:, :out] + m[:, out:2 * out] + m[:, 2 * out:3 * out] + m[:, 3 * out:]
    o_ref[...] = jax.nn.sigmoid(acc * (1.0 / _H) + b_ref[...] + skip_ref[...])


def _dense1(x, Ws, Vs, Vd, Wl):
    f = pl.pallas_call(
        _dense1_body,
        grid=(_NB,),
        in_specs=[
            pl.BlockSpec((_BR, _D), lambda i: (i, 0)),
            pl.BlockSpec((_D, _H * _HID), lambda i: (0, 0)),
            pl.BlockSpec((_D, _H), lambda i: (0, 0)),
            pl.BlockSpec((_D, _H), lambda i: (0, 0)),
            pl.BlockSpec((_D, _HID), lambda i: (0, 0)),
        ],
        out_specs=[
            pl.BlockSpec((_BR, _H * _HID), lambda i: (i, 0)),
            pl.BlockSpec((_BR, _H), lambda i: (i, 0)),
            pl.BlockSpec((_BR, _H), lambda i: (i, 0)),
            pl.BlockSpec((_BR, _HID), lambda i: (i, 0)),
        ],
        out_shape=[
            jax.ShapeDtypeStruct((_N, _H * _HID), jnp.float32),
            jax.ShapeDtypeStruct((_N, _H), jnp.float32),
            jax.ShapeDtypeStruct((_N, _H), jnp.float32),
            jax.ShapeDtypeStruct((_N, _HID), jnp.float32),
        ],
    )
    return f(x, Ws, Vs, Vd, Wl)


def _dense2(msg, skip, b, Ws, Vs, Vd, Wl):
    f = pl.pallas_call(
        _dense2_body,
        grid=(_NB,),
        in_specs=[
            pl.BlockSpec((_BR, _H * _HID), lambda i: (i, 0)),
            pl.BlockSpec((_BR, _HID), lambda i: (i, 0)),
            pl.BlockSpec((1, _HID), lambda i: (0, 0)),
            pl.BlockSpec((_HID, _H * _OUT), lambda i: (0, 0)),
            pl.BlockSpec((_HID, _H), lambda i: (0, 0)),
            pl.BlockSpec((_HID, _H), lambda i: (0, 0)),
            pl.BlockSpec((_HID, _OUT), lambda i: (0, 0)),
        ],
        out_specs=[
            pl.BlockSpec((_BR, _HID), lambda i: (i, 0)),
            pl.BlockSpec((_BR, _H * _OUT), lambda i: (i, 0)),
            pl.BlockSpec((_BR, _H), lambda i: (i, 0)),
            pl.BlockSpec((_BR, _H), lambda i: (i, 0)),
            pl.BlockSpec((_BR, _OUT), lambda i: (i, 0)),
        ],
        out_shape=[
            jax.ShapeDtypeStruct((_N, _HID), jnp.float32),
            jax.ShapeDtypeStruct((_N, _H * _OUT), jnp.float32),
            jax.ShapeDtypeStruct((_N, _H), jnp.float32),
            jax.ShapeDtypeStruct((_N, _H), jnp.float32),
            jax.ShapeDtypeStruct((_N, _OUT), jnp.float32),
        ],
    )
    return f(msg, skip, b.reshape(1, _HID), Ws, Vs, Vd, Wl)


def _final(msg, skip, b):
    f = pl.pallas_call(
        _final_body,
        grid=(_NB,),
        in_specs=[
            pl.BlockSpec((_BR, _H * _OUT), lambda i: (i, 0)),
            pl.BlockSpec((_BR, _OUT), lambda i: (i, 0)),
            pl.BlockSpec((1, _OUT), lambda i: (0, 0)),
        ],
        out_specs=pl.BlockSpec((_BR, _OUT), lambda i: (i, 0)),
        out_shape=jax.ShapeDtypeStruct((_N, _OUT), jnp.float32),
    )
    return f(msg, skip, b.reshape(1, _OUT))


def _edge_phase(xs, es, ed, src, dst):
    gmax = jnp.max(es, axis=0)
    M = jax.nn.leaky_relu(gmax[None, :] + ed, 0.2)
    alpha = jax.nn.leaky_relu(es[src] + ed[dst], 0.2)
    ex = jnp.exp(alpha - M[dst])
    den = jax.ops.segment_sum(ex, dst, num_segments=_N)
    coef = ex / (den[dst] + 1e-16)
    ch = xs.shape[-1] // _H
    msg = xs[src].reshape(_E, _H, ch) * coef[:, :, None]
    out = jax.ops.segment_sum(msg.reshape(_E, _H * ch), dst, num_segments=_N)
    return out


def kernel(x, edge_index, W1s, W1d, a1s, a1d, b1, Wl1, bl1,
           W2s, W2d, a2s, a2d, b2, Wl2, bl2):
    src, dst = edge_index[0], edge_index[1]
    A1s = jax.scipy.linalg.block_diag(*[a1s[h][:, None] for h in range(_H)])
    A1d = jax.scipy.linalg.block_diag(*[a1d[h][:, None] for h in range(_H)])
    A2s = jax.scipy.linalg.block_diag(*[a2s[h][:, None] for h in range(_H)])
    A2d = jax.scipy.linalg.block_diag(*[a2d[h][:, None] for h in range(_H)])
    V1s, V1d = W1s @ A1s, W1d @ A1d
    V2s, V2d = W2s @ A2s, W2d @ A2d

    xs1, es1, ed1, skip1 = _dense1(x, W1s, V1s, V1d, Wl1)
    skip1 = skip1 + bl1[None, :]
    msg1 = _edge_phase(xs1, es1, ed1, src, dst)
    h, xs2, es2, ed2, skip2 = _dense2(msg1, skip1, b1, W2s, V2s, V2d, Wl2)
    skip2 = skip2 + bl2[None, :]
    msg2 = _edge_phase(xs2, es2, ed2, src, dst)
    return _final(msg2, skip2, b2)
